# Initial kernel scaffold; baseline (speedup 1.0000x reference)
import jax
import jax.numpy as jnp
from jax.experimental import pallas as pl


def kernel(qlen, klen, W):
    raise NotImplementedError("write your pallas kernel here")


# speedup vs baseline: 42.4829x; 42.4829x over previous
import functools

import jax
import jax.numpy as jnp
from jax import lax
from jax.experimental import pallas as pl
from jax.experimental.pallas import tpu as pltpu
from jax.experimental.pallas import tpu_sc as plsc

MAX_REL_K = 512
HEADS = 16
SEQ = 2048
TBL = 2 * MAX_REL_K + 1
WROW = 1032
PV_LEN = 4096
LANES = 16
NCORES = 2
NSUB = 16


def _make_sc_kernel():
    mesh = plsc.VectorSubcoreMesh(core_axis_name="c", subcore_axis_name="s")

    @functools.partial(
        pl.kernel,
        mesh=mesh,
        compiler_params=pltpu.CompilerParams(needs_layout_passes=False),
        out_type=jax.ShapeDtypeStruct((HEADS * SEQ * SEQ,), jnp.float32),
        scratch_types=[
            pltpu.VMEM((WROW,), jnp.float32),
        ]
        + [pltpu.VMEM((PV_LEN,), jnp.float32) for _ in range(8)]
        + [
            pltpu.SemaphoreType.DMA,
        ],
    )
    def body(wt_hbm, out_hbm, wh_v, *pv_and_sem):
        pv_v = pv_and_sem[:8]
        sem = pv_and_sem[8]
        wid = lax.axis_index("s") * NCORES + lax.axis_index("c")
        h = wid // 2
        i0 = (wid % 2) * (SEQ // 2)

        pltpu.sync_copy(wt_hbm.at[h], wh_v)

        lane = lax.iota(jnp.int32, LANES)
        for r in range(8):
            def build(c, _, r=r):
                u = c * LANES + lane + (r - (SEQ - 1))
                idx = jnp.clip(u, -MAX_REL_K, MAX_REL_K) + MAX_REL_K
                pv_v[r][pl.ds(c * LANES, LANES)] = plsc.load_gather(wh_v, [idx])
                return 0

            lax.fori_loop(0, PV_LEN // LANES, build, 0)

        def group(k, _):
            q = (SEQ - 8) - i0 - 8 * k
            cps = [
                pltpu.async_copy(
                    pv_v[7 - p].at[pl.ds(q, SEQ)],
                    out_hbm.at[pl.ds((h * SEQ + i0 + 8 * k + p) * SEQ, SEQ)],
                    sem,
                )
                for p in range(8)
            ]
            for cp in cps:
                cp.wait()
            return 0

        lax.fori_loop(0, SEQ // 2 // 8, group, 0)

    return body


_sc_bias = _make_sc_kernel()


def kernel(qlen, klen, W):
    wt = jnp.zeros((HEADS, WROW), jnp.float32).at[:, :TBL].set(W.T)
    return _sc_bias(wt).reshape(HEADS, SEQ, SEQ)
